# Initial kernel scaffold; baseline (speedup 1.0000x reference)
import jax
import jax.numpy as jnp
from jax.experimental import pallas as pl


def kernel(results, features, feature_weights, indexes, labels, cur_epoch):
    raise NotImplementedError("write your pallas kernel here")


# speedup vs baseline: 10.3010x; 10.3010x over previous
import functools

import jax
import jax.numpy as jnp
from jax import lax
from jax.experimental import pallas as pl
from jax.experimental.pallas import tpu as pltpu
from jax.experimental.pallas import tpu_sc as plsc

_TEMP = 0.05
_M = 100000
_F = 128
_C = 1000
_CP = 1024
_B = 1024
_NC = 2
_NS = 16
_NW = _NC * _NS
_CH = 128
_NFULL = _M // _CH
_TAIL = _M - _NFULL * _CH
_TAIL_BASE = _NFULL * _CH
_KMAX = (_NFULL + _NW - 1) // _NW
_RPW = _B // _NW
_SLICE = _CP // _NS


_MPAD = ((_M + _F - 1) // _F) * _F


def _sc_segment_sum(features, labels, labels2d, indexes):
    mesh = plsc.VectorSubcoreMesh(
        core_axis_name="c", subcore_axis_name="s",
        num_cores=_NC, num_subcores=_NS)

    @functools.partial(
        pl.kernel,
        out_type=(
            jax.ShapeDtypeStruct((_NC, _CP, _F), jnp.float32),
            jax.ShapeDtypeStruct((_NC, _CP, 16), jnp.float32),
            jax.ShapeDtypeStruct((_B, _F), jnp.int32),
        ),
        mesh=mesh,
        scratch_types=[
            pltpu.VMEM((_CH, _F), jnp.float32),
            pltpu.VMEM((_CH,), jnp.int32),
            pltpu.VMEM((_CH, 16), jnp.float32),
            pltpu.VMEM((_TAIL, _F), jnp.float32),
            pltpu.VMEM((_TAIL,), jnp.int32),
            pltpu.VMEM((_TAIL, 16), jnp.float32),
            pltpu.VMEM((_SLICE, _F), jnp.float32),
            pltpu.VMEM((_SLICE, 16), jnp.float32),
            pltpu.VMEM((_RPW,), jnp.int32),
            pltpu.VMEM((_RPW,), jnp.int32),
            pltpu.VMEM((_RPW, _F), jnp.int32),
            pltpu.VMEM_SHARED((_CP, _F), jnp.float32),
            pltpu.VMEM_SHARED((_CP, 16), jnp.float32),
            pltpu.SemaphoreType.DMA,
        ],
    )
    def k(features_h, labels_h, labels2d_h, indexes_h,
          cf_out, nums_out, tgt_out,
          fbuf, lbuf, ones_v, fbuf_t, lbuf_t, ones_t, zbuf, znums,
          idxb, ridxb, gbuf, cf_acc, nums_acc, sem):
        cid = lax.axis_index("c")
        sid = lax.axis_index("s")
        w = cid * _NS + sid

        zrow = jnp.zeros((16,), jnp.float32)
        orow = jnp.ones((16,), jnp.float32)

        def fill_zero(i, _):
            def inner(j, _):
                zbuf[i, pl.ds(j * 16, 16)] = zrow
                return 0
            lax.fori_loop(0, _F // 16, inner, 0)
            znums[i, :] = zrow
            return 0
        lax.fori_loop(0, _SLICE, fill_zero, 0)

        def fill_ones(i, _):
            ones_v[i, :] = orow
            return 0
        lax.fori_loop(0, _CH, fill_ones, 0)

        def fill_ones_t(i, _):
            ones_t[i, :] = orow
            return 0
        lax.fori_loop(0, _TAIL, fill_ones_t, 0)

        base = sid * _SLICE
        pltpu.sync_copy(zbuf, cf_acc.at[pl.ds(base, _SLICE)])
        pltpu.sync_copy(znums, nums_acc.at[pl.ds(base, _SLICE)])
        plsc.subcore_barrier()

        def chunk_body(kk, _):
            c = w + kk * _NW

            @pl.when(c < _NFULL)
            def _():
                off = c * _CH
                pltpu.sync_copy(features_h.at[pl.ds(off, _CH)], fbuf)
                pltpu.sync_copy(labels_h.at[pl.ds(off, _CH)], lbuf)
                pltpu.sync_copy(fbuf, cf_acc.at[lbuf], add=True)
                pltpu.sync_copy(ones_v, nums_acc.at[lbuf], add=True)
            return 0
        lax.fori_loop(0, _KMAX, chunk_body, 0)

        @pl.when(w == _NW - 1)
        def _():
            pltpu.sync_copy(features_h.at[pl.ds(_TAIL_BASE, _TAIL)], fbuf_t)
            pltpu.sync_copy(labels_h.at[pl.ds(_TAIL_BASE, _TAIL)], lbuf_t)
            pltpu.sync_copy(fbuf_t, cf_acc.at[lbuf_t], add=True)
            pltpu.sync_copy(ones_t, nums_acc.at[lbuf_t], add=True)

        tb = w * _RPW
        pltpu.sync_copy(indexes_h.at[pl.ds(tb, _RPW)], idxb)

        def rid_body(j, _):
            v = idxb[pl.ds(j * 16, 16)]
            ridxb[pl.ds(j * 16, 16)] = lax.shift_right_logical(v, 7)
            return 0
        lax.fori_loop(0, _RPW // 16, rid_body, 0)
        pltpu.async_copy(labels2d_h.at[ridxb], gbuf, sem).wait()
        pltpu.sync_copy(gbuf, tgt_out.at[pl.ds(tb, _RPW)])

        plsc.subcore_barrier()
        pltpu.sync_copy(cf_acc.at[pl.ds(base, _SLICE)],
                        cf_out.at[cid, pl.ds(base, _SLICE)])
        pltpu.sync_copy(nums_acc.at[pl.ds(base, _SLICE)],
                        nums_out.at[cid, pl.ds(base, _SLICE)])

    return k(features, labels, labels2d, indexes)


def _tc_loss_body(res_ref, cf_ref, nums_ref, trows_ref, idx_ref, out_ref):
    r = res_ref[:]
    nrm = jnp.sqrt(jnp.sum(r * r, axis=1, keepdims=True))
    inputs = r / jnp.clip(nrm, 1e-12, None)
    cf = cf_ref[0] + cf_ref[1]
    nums = nums_ref[0, :, 0:1] + nums_ref[1, :, 0:1]
    sim = lax.dot_general(
        cf, inputs, (((1,), (1,)), ((), ())),
        preferred_element_type=jnp.float32,
        precision=lax.Precision.HIGHEST)
    mask = (nums > 0).astype(jnp.float32)
    denom = mask * nums + (1.0 - mask)
    vec = sim / _TEMP / denom
    exps = jnp.exp(vec)
    masked = exps * mask
    sums = jnp.sum(masked, axis=0, keepdims=True) + 1e-6
    msim = masked / sums
    logp = jnp.log(msim + 1e-6)
    col = jnp.bitwise_and(idx_ref[:], _F - 1)
    lane = lax.broadcasted_iota(jnp.int32, (_B, _F), 1)
    tgt = jnp.sum(jnp.where(lane == col, trows_ref[:], 0),
                  axis=1, keepdims=True)
    oh = (lax.broadcasted_iota(jnp.int32, (_B, _CP), 1)
          == tgt).astype(jnp.float32)
    m = lax.dot_general(
        oh, logp, (((1,), (0,)), ((), ())),
        preferred_element_type=jnp.float32,
        precision=lax.Precision.HIGHEST)
    eye = (lax.broadcasted_iota(jnp.int32, (_B, _B), 0)
           == lax.broadcasted_iota(jnp.int32, (_B, _B), 1))
    out_ref[...] = jnp.reshape(-jnp.sum(jnp.where(eye, m, 0.0)) / _B, (1, 1))


def _tc_loss(results, cf_parts, nums_parts, trows, idx2):
    return pl.pallas_call(
        _tc_loss_body,
        out_shape=jax.ShapeDtypeStruct((1, 1), jnp.float32),
    )(results, cf_parts, nums_parts, trows, idx2)


def kernel(results, features, feature_weights, indexes, labels, cur_epoch):
    del feature_weights, cur_epoch
    labels_i = labels.astype(jnp.int32)
    idx_i = indexes.astype(jnp.int32)
    labels2d = jnp.concatenate(
        [labels_i, jnp.zeros((_MPAD - _M,), jnp.int32)]).reshape(_MPAD // _F, _F)
    cf_parts, nums_parts, trows = _sc_segment_sum(
        features, labels_i, labels2d, idx_i)
    out = _tc_loss(results, cf_parts, nums_parts, trows,
                   idx_i.reshape(_B, 1))
    return out[0, 0]


# speedup vs baseline: 17.0366x; 1.6539x over previous
import functools

import jax
import jax.numpy as jnp
from jax import lax
from jax.experimental import pallas as pl
from jax.experimental.pallas import tpu as pltpu
from jax.experimental.pallas import tpu_sc as plsc

_TEMP = 0.05
_M = 100000
_F = 128
_C = 1000
_CP = 1024
_B = 1024
_NC = 2
_NS = 16
_NW = _NC * _NS
_CH = 128
_NFULL = _M // _CH
_TAIL = _M - _NFULL * _CH
_TAIL_BASE = _NFULL * _CH
_KMAX = (_NFULL + _NW - 1) // _NW
_RPW = _B // _NW
_SLICE = _CP // _NS
_MPAD = ((_M + _F - 1) // _F) * _F


def _sc_segment_sum(features, labels, labels2d, indexes):
    mesh = plsc.VectorSubcoreMesh(
        core_axis_name="c", subcore_axis_name="s",
        num_cores=_NC, num_subcores=_NS)

    @functools.partial(
        pl.kernel,
        out_type=(
            jax.ShapeDtypeStruct((_NC, _CP, _F), jnp.float32),
            jax.ShapeDtypeStruct((_NC, _CP, 16), jnp.float32),
            jax.ShapeDtypeStruct((_B, _F), jnp.int32),
        ),
        mesh=mesh,
        scratch_types=[
            pltpu.VMEM((_CH, _F), jnp.float32),
            pltpu.VMEM((_CH, _F), jnp.float32),
            pltpu.VMEM((_CH,), jnp.int32),
            pltpu.VMEM((_CH,), jnp.int32),
            pltpu.VMEM((_CH, 16), jnp.float32),
            pltpu.VMEM((_TAIL, _F), jnp.float32),
            pltpu.VMEM((_TAIL,), jnp.int32),
            pltpu.VMEM((_TAIL, 16), jnp.float32),
            pltpu.VMEM((_SLICE, _F), jnp.float32),
            pltpu.VMEM((_SLICE, 16), jnp.float32),
            pltpu.VMEM((_RPW,), jnp.int32),
            pltpu.VMEM((_RPW,), jnp.int32),
            pltpu.VMEM((_RPW, _F), jnp.int32),
            pltpu.VMEM_SHARED((_CP, _F), jnp.float32),
            pltpu.VMEM_SHARED((_CP, 16), jnp.float32),
            pltpu.SemaphoreType.DMA,
            pltpu.SemaphoreType.DMA,
            pltpu.SemaphoreType.DMA,
            pltpu.SemaphoreType.DMA,
            pltpu.SemaphoreType.DMA,
        ],
    )
    def k(features_h, labels_h, labels2d_h, indexes_h,
          cf_out, nums_out, tgt_out,
          fbuf0, fbuf1, lbuf0, lbuf1, ones_v, fbuf_t, lbuf_t, ones_t,
          zbuf, znums, idxb, ridxb, gbuf, cf_acc, nums_acc,
          sem_r0, sem_r1, sem_s0, sem_s1, sem_t):
        cid = lax.axis_index("c")
        sid = lax.axis_index("s")
        w = cid * _NS + sid
        fbufs = (fbuf0, fbuf1)
        lbufs = (lbuf0, lbuf1)
        sem_r = (sem_r0, sem_r1)
        sem_s = (sem_s0, sem_s1)

        def issue_read(k_slot):
            b = k_slot % 2
            c = w + k_slot * _NW
            c = jnp.where(c < _NFULL, c, 0)
            off = pl.multiple_of(c * _CH, _CH)
            rf = pltpu.async_copy(
                features_h.at[pl.ds(off, _CH)], fbufs[b], sem_r[b])
            rl = pltpu.async_copy(
                labels_h.at[pl.ds(off, _CH)], lbufs[b], sem_r[b])
            return rf, rl

        def issue_scatter(b):
            sf = pltpu.async_copy(
                fbufs[b], cf_acc.at[lbufs[b]], sem_s[b], add=True)
            so = pltpu.async_copy(
                ones_v, nums_acc.at[lbufs[b]], sem_s[b], add=True)
            return sf, so

        rd = [None, None]
        sc = [None, None]
        rd[0] = issue_read(0)

        tb = w * _RPW
        pltpu.async_copy(indexes_h.at[pl.ds(tb, _RPW)], idxb, sem_t).wait()

        def rid_body(j, _):
            v = idxb[pl.ds(j * 16, 16)]
            ridxb[pl.ds(j * 16, 16)] = lax.shift_right_logical(v, 7)
            return 0
        lax.fori_loop(0, _RPW // 16, rid_body, 0)
        pltpu.async_copy(labels2d_h.at[ridxb], gbuf, sem_t).wait()
        pltpu.sync_copy(gbuf, tgt_out.at[pl.ds(tb, _RPW)])

        zrow = jnp.zeros((16,), jnp.float32)
        orow = jnp.ones((16,), jnp.float32)

        def fill_zero(i, _):
            def inner(j, _):
                zbuf[i, pl.ds(j * 16, 16)] = zrow
                return 0
            lax.fori_loop(0, _F // 16, inner, 0)
            znums[i, :] = zrow
            return 0
        lax.fori_loop(0, _SLICE, fill_zero, 0)

        def fill_ones(i, _):
            ones_v[i, :] = orow
            return 0
        lax.fori_loop(0, _CH, fill_ones, 0)

        def fill_ones_t(i, _):
            ones_t[i, :] = orow
            return 0
        lax.fori_loop(0, _TAIL, fill_ones_t, 0)

        base = sid * _SLICE
        pltpu.sync_copy(zbuf, cf_acc.at[pl.ds(base, _SLICE)])
        pltpu.sync_copy(znums, nums_acc.at[pl.ds(base, _SLICE)])
        plsc.subcore_barrier()

        for ks in range(_KMAX):
            b = ks % 2
            if ks + 1 < _KMAX:
                if ks >= 1:
                    sc[1 - b][0].wait()
                    sc[1 - b][1].wait()
                rd[1 - b] = issue_read(ks + 1)
            rd[b][0].wait()
            rd[b][1].wait()
            if ks == _KMAX - 1:
                valid = (w + ks * _NW) < _NFULL
                for j in range(_CH // 16):
                    v = lbufs[b][pl.ds(j * 16, 16)]
                    pad = jnp.full((16,), _C, jnp.int32)
                    lbufs[b][pl.ds(j * 16, 16)] = jnp.where(valid, v, pad)
            sc[b] = issue_scatter(b)

        sc[(_KMAX - 2) % 2][0].wait()
        sc[(_KMAX - 2) % 2][1].wait()
        sc[(_KMAX - 1) % 2][0].wait()
        sc[(_KMAX - 1) % 2][1].wait()

        @pl.when(w == _NW - 1)
        def _():
            pltpu.sync_copy(features_h.at[pl.ds(_TAIL_BASE, _TAIL)], fbuf_t)
            pltpu.sync_copy(labels_h.at[pl.ds(_TAIL_BASE, _TAIL)], lbuf_t)
            pltpu.sync_copy(fbuf_t, cf_acc.at[lbuf_t], add=True)
            pltpu.sync_copy(ones_t, nums_acc.at[lbuf_t], add=True)

        plsc.subcore_barrier()
        pltpu.sync_copy(cf_acc.at[pl.ds(base, _SLICE)],
                        cf_out.at[cid, pl.ds(base, _SLICE)])
        pltpu.sync_copy(nums_acc.at[pl.ds(base, _SLICE)],
                        nums_out.at[cid, pl.ds(base, _SLICE)])

    return k(features, labels, labels2d, indexes)


def _tc_loss_body(res_ref, cf_ref, nums_ref, trows_ref, idx_ref, out_ref):
    r = res_ref[:]
    nrm = jnp.sqrt(jnp.sum(r * r, axis=1, keepdims=True))
    inputs = r / jnp.clip(nrm, 1e-12, None)
    cf = cf_ref[0] + cf_ref[1]
    nums = nums_ref[0, :, 0:1] + nums_ref[1, :, 0:1]
    sim = lax.dot_general(
        cf, inputs, (((1,), (1,)), ((), ())),
        preferred_element_type=jnp.float32,
        precision=lax.Precision.HIGHEST)
    valid_c = lax.broadcasted_iota(jnp.int32, (_CP, 1), 0) < _C
    mask = ((nums > 0) & valid_c).astype(jnp.float32)
    denom = mask * nums + (1.0 - mask)
    vec = sim / _TEMP / denom
    vec = jnp.where(mask > 0, vec, 0.0)
    exps = jnp.exp(vec)
    masked = exps * mask
    sums = jnp.sum(masked, axis=0, keepdims=True) + 1e-6
    msim = masked / sums
    logp = jnp.log(msim + 1e-6)
    col = jnp.bitwise_and(idx_ref[:], _F - 1)
    lane = lax.broadcasted_iota(jnp.int32, (_B, _F), 1)
    tgt = jnp.sum(jnp.where(lane == col, trows_ref[:], 0),
                  axis=1, keepdims=True).astype(jnp.float32)
    eye = (lax.broadcasted_iota(jnp.int32, (_B, _B), 0)
           == lax.broadcasted_iota(jnp.int32, (_B, _B), 1)
           ).astype(jnp.float32)
    tgt_row = lax.dot_general(
        tgt, eye, (((0,), (0,)), ((), ())),
        preferred_element_type=jnp.float32,
        precision=lax.Precision.HIGHEST)
    cidx = lax.broadcasted_iota(jnp.int32, (_CP, _B), 0)
    picked = jnp.where(cidx == tgt_row.astype(jnp.int32), logp, 0.0)
    out_ref[...] = jnp.reshape(-jnp.sum(picked) / _B, (1, 1))


def _tc_loss(results, cf_parts, nums_parts, trows, idx2):
    return pl.pallas_call(
        _tc_loss_body,
        out_shape=jax.ShapeDtypeStruct((1, 1), jnp.float32),
    )(results, cf_parts, nums_parts, trows, idx2)


def kernel(results, features, feature_weights, indexes, labels, cur_epoch):
    del feature_weights, cur_epoch
    labels_i = labels.astype(jnp.int32)
    idx_i = indexes.astype(jnp.int32)
    labels2d = jnp.concatenate(
        [labels_i, jnp.zeros((_MPAD - _M,), jnp.int32)]).reshape(_MPAD // _F, _F)
    cf_parts, nums_parts, trows = _sc_segment_sum(
        features, labels_i, labels2d, idx_i)
    out = _tc_loss(results, cf_parts, nums_parts, trows,
                   idx_i.reshape(_B, 1))
    return out[0, 0]
